# baseline (speedup 1.0000x reference)
import jax
import jax.numpy as jnp
from jax import lax
from jax.experimental import pallas as pl
from jax.experimental.pallas import tpu as pltpu
from jax.experimental.pallas import tpu_sc as plsc

_FFT = 4096
_NSC = 3276

_NC = 2
_NS = 16
_NW = _NC * _NS

_RB = 8
_IN_W = _RB * _FFT
_OUT_W = _RB * _NSC
_NVEC = _OUT_W // 16
_UNROLL = 26


def _body(x_hbm, idx_hbm, out_hbm, idxbuf, inbuf, outbuf):
    wid = lax.axis_index("s") * _NC + lax.axis_index("c")
    nblk = x_hbm.shape[0] // _IN_W // _NW
    blk0 = wid * nblk
    pltpu.sync_copy(idx_hbm, idxbuf)

    def do_block(j, _):
        blk = blk0 + j
        pltpu.sync_copy(x_hbm.at[pl.ds(blk * _IN_W, _IN_W)], inbuf)

        def do_vec(k, _):
            for i in range(_UNROLL):
                off = pl.multiple_of((k * _UNROLL + i) * 16, 16)
                iv = idxbuf[pl.ds(off, 16)]
                outbuf[pl.ds(off, 16)] = plsc.load_gather(inbuf, [iv])
            return 0

        lax.fori_loop(0, _NVEC // _UNROLL, do_vec, 0, unroll=False)
        pltpu.sync_copy(outbuf, out_hbm.at[pl.ds(blk * _OUT_W, _OUT_W)])
        return 0

    lax.fori_loop(0, nblk, do_block, 0, unroll=False)


def kernel(inputs, sc_ind):
    lead = inputs.shape[:-1]
    rows = 1
    for d in lead:
        rows *= d
    x = inputs.reshape(rows * _FFT)
    flat_idx = (jnp.arange(_RB, dtype=jnp.int32)[:, None] * _FFT
                + sc_ind[None, :].astype(jnp.int32)).reshape(-1)
    mesh = plsc.VectorSubcoreMesh(core_axis_name="c", subcore_axis_name="s")
    out = pl.kernel(
        _body,
        out_type=jax.ShapeDtypeStruct((rows * _NSC,), inputs.dtype),
        mesh=mesh,
        scratch_types=[pltpu.VMEM((_OUT_W,), jnp.int32),
                       pltpu.VMEM((_IN_W,), jnp.float32),
                       pltpu.VMEM((_OUT_W,), jnp.float32)],
        compiler_params=pltpu.CompilerParams(use_tc_tiling_on_sc=False,
                                             needs_layout_passes=False),
    )(x, flat_idx)
    return out.reshape(*lead, _NSC)


# speedup vs baseline: 1.1354x; 1.1354x over previous
import jax
import jax.numpy as jnp
from jax import lax
from jax.experimental import pallas as pl
from jax.experimental.pallas import tpu as pltpu
from jax.experimental.pallas import tpu_sc as plsc

_FFT = 4096
_NSC = 3276
_ROWS = 14
_COL0 = 384
_NCOL = 3328

_NC = 2
_NS = 16
_NW = _NC * _NS

_NVEC = 205
_UNROLL = 5


def _body(x_hbm, ctab_hbm, out_hbm, ctab, inbuf, outbuf):
    wid = lax.axis_index("s") * _NC + lax.axis_index("c")
    nsl = x_hbm.shape[0] // _NW
    s0 = wid * nsl
    pltpu.sync_copy(ctab_hbm, ctab)
    iota = lax.iota(jnp.int32, 16)
    colmax = jnp.full((16,), _NSC - 1, jnp.int32)

    for j in range(nsl):
        sl = s0 + j
        pltpu.sync_copy(x_hbm.at[sl, :, pl.ds(_COL0, _NCOL)], inbuf)

        def do_row(r, _):
            rowv = jnp.full((16,), 0, jnp.int32) + r

            def do_vec(k, _):
                for i in range(_UNROLL):
                    off = pl.multiple_of((k * _UNROLL + i) * 16, 16)
                    cin = ctab[pl.ds(off, 16)]
                    v = plsc.load_gather(inbuf, [rowv, cin])
                    cout = jnp.minimum(iota + off, colmax)
                    plsc.store_scatter(outbuf, [rowv, cout], v)
                return 0

            lax.fori_loop(0, _NVEC // _UNROLL, do_vec, 0, unroll=False)
            return 0

        lax.fori_loop(0, _ROWS, do_row, 0, unroll=False)
        pltpu.sync_copy(outbuf, out_hbm.at[sl])


def kernel(inputs, sc_ind):
    lead = inputs.shape[:-1]
    nsl = 1
    for d in lead[:-1]:
        nsl *= d
    x = inputs.reshape(nsl, _ROWS, _FFT)
    ctab = jnp.pad(sc_ind.astype(jnp.int32) - _COL0, (0, _NVEC * 16 - _NSC),
                   mode="edge")
    mesh = plsc.VectorSubcoreMesh(core_axis_name="c", subcore_axis_name="s")
    out = pl.kernel(
        _body,
        out_type=jax.ShapeDtypeStruct((nsl, _ROWS, _NSC), inputs.dtype),
        mesh=mesh,
        scratch_types=[pltpu.VMEM((_NVEC * 16,), jnp.int32),
                       pltpu.VMEM((_ROWS, _NCOL), jnp.float32),
                       pltpu.VMEM((_ROWS, _NSC), jnp.float32)],
        compiler_params=pltpu.CompilerParams(use_tc_tiling_on_sc=True,
                                             needs_layout_passes=False),
    )(x, ctab)
    return out.reshape(*lead, _NSC)
